# baseline (speedup 1.0000x reference)
import functools

import jax
import jax.numpy as jnp
from jax import lax
from jax.experimental import pallas as pl
from jax.experimental.pallas import tpu as pltpu
from jax.experimental.pallas import tpu_sc as plsc

N_NODES = 10000
N_EDGES = 320000
D = 128

NC = 2
NS = 16
NW = NC * NS
NROW = 10240
CH = 125
HCH = 40
NCH = 2 * HCH
EPW = NCH * CH
RPS = NROW // NS


def _sc_agg_body(edge_hbm, x_hbm, out_hbm,
                 srcv, dstv, rows_a, rows_b, aggsh, sem_a, sem_b):
    cid = lax.axis_index("c")
    sid = lax.axis_index("s")
    wid = cid * NS + sid

    def _zrow(r, carry):
        for c in range(D // 16):
            rows_a[r, pl.ds(c * 16, 16)] = jnp.zeros((16,), jnp.float32)
        return carry
    lax.fori_loop(0, 80, _zrow, 0)
    for z in range(RPS // 80):
        pltpu.sync_copy(rows_a.at[pl.ds(0, 80)],
                        aggsh.at[pl.ds(sid * RPS + z * 80, 80)])

    plsc.subcore_barrier()

    for h in range(2):
        pltpu.sync_copy(edge_hbm.at[0, wid, h], srcv)
        pltpu.sync_copy(edge_hbm.at[1, wid, h], dstv)

        pltpu.async_copy(x_hbm.at[srcv.at[0]], rows_a, sem_a)

        def _pair(p, carry):
            i = 2 * p
            pltpu.async_copy(x_hbm.at[srcv.at[i + 1]], rows_b, sem_b)
            pltpu.make_async_copy(
                x_hbm.at[srcv.at[i]], rows_a, sem_a).wait()
            pltpu.sync_copy(rows_a, aggsh.at[dstv.at[i]], add=True)
            pltpu.async_copy(x_hbm.at[srcv.at[i + 2]], rows_a, sem_a)
            pltpu.make_async_copy(
                x_hbm.at[srcv.at[i + 1]], rows_b, sem_b).wait()
            pltpu.sync_copy(rows_b, aggsh.at[dstv.at[i + 1]], add=True)
            return carry

        lax.fori_loop(0, HCH // 2 - 1, _pair, 0)
        pltpu.async_copy(x_hbm.at[srcv.at[HCH - 1]], rows_b, sem_b)
        pltpu.make_async_copy(
            x_hbm.at[srcv.at[HCH - 2]], rows_a, sem_a).wait()
        pltpu.sync_copy(rows_a, aggsh.at[dstv.at[HCH - 2]], add=True)
        pltpu.make_async_copy(
            x_hbm.at[srcv.at[HCH - 1]], rows_b, sem_b).wait()
        pltpu.sync_copy(rows_b, aggsh.at[dstv.at[HCH - 1]], add=True)

    plsc.subcore_barrier()
    pltpu.sync_copy(aggsh.at[pl.ds(sid * RPS, RPS)],
                    out_hbm.at[pl.ds(cid * NROW + sid * RPS, RPS)])


_sc_agg = functools.partial(
    pl.kernel,
    out_type=jax.ShapeDtypeStruct((NC * NROW, D), jnp.float32),
    mesh=plsc.VectorSubcoreMesh(core_axis_name="c", subcore_axis_name="s"),
    scratch_types=[
        pltpu.VMEM((HCH, CH), jnp.int32),
        pltpu.VMEM((HCH, CH), jnp.int32),
        pltpu.VMEM((CH, D), jnp.float32),
        pltpu.VMEM((CH, D), jnp.float32),
        pltpu.VMEM_SHARED((NROW, D), jnp.float32),
        pltpu.SemaphoreType.DMA,
        pltpu.SemaphoreType.DMA,
    ],
    name="sc_edge_segment_sum",
)(_sc_agg_body)


def _mlp_body(x_ref, part_ref, w1a_ref, w1b_ref, b1_ref,
              gamma_ref, beta_ref, w2_ref, b2_ref, out_ref):
    agg = part_ref[:N_NODES] + part_ref[NROW:NROW + N_NODES]
    h = jnp.dot(x_ref[...], w1a_ref[...], preferred_element_type=jnp.float32)
    h = h + jnp.dot(agg, w1b_ref[...], preferred_element_type=jnp.float32)
    h = jnp.maximum(h + b1_ref[...], 0.0)
    mean = jnp.mean(h, axis=0, keepdims=True)
    cen = h - mean
    var = jnp.mean(cen * cen, axis=0, keepdims=True)
    hn = cen * (lax.rsqrt(var + 1e-5) * gamma_ref[...]) + beta_ref[...]
    out_ref[...] = (
        jnp.dot(hn, w2_ref[...], preferred_element_type=jnp.float32)
        + b2_ref[...])


def kernel(x, edge_index, W1, b1, gamma, beta, W2, b2):
    edges = edge_index.reshape(2, NW, 2, HCH, CH)
    partials = _sc_agg(edges, x)
    return pl.pallas_call(
        _mlp_body,
        out_shape=jax.ShapeDtypeStruct((N_NODES, D), jnp.float32),
    )(x, partials, W1[:D], W1[D:], b1.reshape(1, D),
      gamma.reshape(1, D), beta.reshape(1, D), W2, b2.reshape(1, D))
